# baseline (speedup 1.0000x reference)
import functools
import jax
import jax.numpy as jnp
import numpy as np
from jax import lax
from jax.experimental import pallas as pl
from jax.experimental.pallas import tpu as pltpu

B = 4; L = 2048; ENC_IN = 21; C_OUT = 21; D = 1024; H = 16; DH = D // H
DFF = 2048; NL = 2; NH = 4; BS = 4; TF = 4
NBK = L // BS
M = B * L
MT = 256
NT = L // 128
R4 = B * H * NH


def _layernorm(y, g, b):
    m = jnp.mean(y, axis=-1, keepdims=True)
    c = y - m
    v = jnp.mean(c * c, axis=-1, keepdims=True)
    return c / jnp.sqrt(v + 1e-5) * g + b


def _mm_body(x_ref, w_ref, o_ref):
    x = x_ref[...]
    o_ref[:, 0:D] = jnp.dot(x, w_ref[:, 0:D],
                            preferred_element_type=jnp.float32)
    o_ref[:, D:2 * D] = jnp.dot(x, w_ref[:, D:2 * D],
                                preferred_element_type=jnp.float32)


def _qkv(x, w):
    return pl.pallas_call(
        _mm_body,
        grid=(M // MT,),
        in_specs=[
            pl.BlockSpec((MT, D), lambda i: (i, 0)),
            pl.BlockSpec((D, 2 * D), lambda i: (0, 0)),
        ],
        out_specs=pl.BlockSpec((MT, 2 * D), lambda i: (i, 0)),
        out_shape=jax.ShapeDtypeStruct((M, 2 * D), jnp.float32),
    )(x, w)


def _bucket_body(qk_ref, rot_ref, o_ref):
    qk = qk_ref[0, 0]
    rv = jnp.dot(qk, rot_ref[...],
                 preferred_element_type=jnp.float32)
    outs = []
    nb2 = NBK // 2
    iota = lax.broadcasted_iota(jnp.int32, (L, nb2), 1)
    big = jnp.int32(1 << 30)
    for r in range(NH):
        seg = rv[:, r * nb2:(r + 1) * nb2]
        mx = jnp.max(seg, axis=1, keepdims=True)
        i1 = jnp.min(jnp.where(seg == mx, iota, big), axis=1)
        mn = jnp.min(seg, axis=1, keepdims=True)
        i2 = jnp.min(jnp.where(seg == mn, iota, big), axis=1)
        bkt = jnp.where(mx[:, 0] >= -mn[:, 0], i1, nb2 + i2)
        outs.append(bkt)
    o_ref[0, 0] = jnp.stack(outs, axis=0)


def _buckets(qkh, rot):
    return pl.pallas_call(
        _bucket_body,
        grid=(B, H),
        in_specs=[
            pl.BlockSpec((1, 1, L, DH), lambda b, h: (b, h, 0, 0)),
            pl.BlockSpec((DH, NH * (NBK // 2)), lambda b, h: (0, 0)),
        ],
        out_specs=pl.BlockSpec((1, 1, NH, L), lambda b, h: (b, h, 0, 0)),
        out_shape=jax.ShapeDtypeStruct((B, H, NH, L), jnp.int32),
    )(qkh, rot)


def _attn_body(qk_ref, kn_ref, v_ref, o_ref, lse_ref):
    scale = DH ** -0.5
    qi = lax.broadcasted_iota(jnp.int32, (128, 256), 0)
    kj = lax.broadcasted_iota(jnp.int32, (128, 256), 1)
    validc = (kj < 128) & ((kj // 4) == (qi // 4))
    validp = (kj >= 128) & (((kj - 128) // 4) == (qi // 4))
    valid = validc | validp
    self_m = kj == qi

    def tile(i, _):
        j = lax.rem(i + NT - 1, NT)
        q = qk_ref[0, pl.ds(i * 128, 128), :]
        kc = kn_ref[0, pl.ds(i * 128, 128), :]
        kp = kn_ref[0, pl.ds(j * 128, 128), :]
        ksh = jnp.concatenate([kp[124:128], kc[0:124]], axis=0)
        kwin = jnp.concatenate([kc, ksh], axis=0)
        vc = v_ref[0, pl.ds(i * 128, 128), :]
        vp = v_ref[0, pl.ds(j * 128, 128), :]
        vsh = jnp.concatenate([vp[124:128], vc[0:124]], axis=0)
        vwin = jnp.concatenate([vc, vsh], axis=0)
        dots = lax.dot_general(q, kwin, (((1,), (1,)), ((), ())),
                               preferred_element_type=jnp.float32) * scale
        dots = jnp.where(self_m, -1e5, dots)
        dots = jnp.where(valid, dots, -1e30)
        mx = jnp.max(dots, axis=1, keepdims=True)
        e = jnp.exp(dots - mx)
        base = (qi // 4) * 4
        w = []
        for t in range(8):
            col = base + t if t < 4 else 128 + base + (t - 4)
            w.append(jnp.sum(jnp.where(kj == col, e, 0.0),
                             axis=1, keepdims=True))
        s = ((w[0] + w[1]) + (w[2] + w[3])) + ((w[4] + w[5]) + (w[6] + w[7]))
        lse = mx + jnp.log(s)
        p = jnp.exp(dots - lse)
        o = jnp.dot(p, vwin, preferred_element_type=jnp.float32)
        o_ref[0, pl.ds(i * 128, 128), :] = o
        lse_ref[0, pl.ds(i * 128, 128), :] = lse
        return 0

    lax.fori_loop(0, NT, tile, 0, unroll=False)


def _attention(sqk, skn, sv):
    bs = lambda: pl.BlockSpec((1, L, DH), lambda i: (i, 0, 0))
    return pl.pallas_call(
        _attn_body,
        grid=(R4,),
        in_specs=[bs(), bs(), bs()],
        out_specs=[
            pl.BlockSpec((1, L, DH), lambda i: (i, 0, 0)),
            pl.BlockSpec((1, L, 1), lambda i: (i, 0, 0)),
        ],
        out_shape=[
            jax.ShapeDtypeStruct((R4, L, DH), jnp.float32),
            jax.ShapeDtypeStruct((R4, L, 1), jnp.float32),
        ],
    )(sqk, skn, sv)


def _mmb_body(x_ref, w_ref, b_ref, o_ref):
    o_ref[...] = jnp.dot(x_ref[...], w_ref[...],
                         preferred_element_type=jnp.float32) + b_ref[...]


def _mm_bias(x, w, b):
    kin, n = w.shape
    vec = lambda i: (0, 0)
    return pl.pallas_call(
        _mmb_body,
        grid=(M // MT,),
        in_specs=[
            pl.BlockSpec((MT, kin), lambda i: (i, 0)),
            pl.BlockSpec((kin, n), vec),
            pl.BlockSpec((1, n), vec),
        ],
        out_specs=pl.BlockSpec((MT, n), lambda i: (i, 0)),
        out_shape=jax.ShapeDtypeStruct((M, n), jnp.float32),
    )(x, w, b.reshape(1, n))


@jax.jit
def _forward_impl(batch_x, batch_x_time_stamp, conv_w, temp_w, Wqk, Wv, Wo,
                  bo, ln1_g, ln1_b, W1, b1, W2, b2, ln2_g, ln2_b, lnf_g,
                  lnf_b, proj_w, proj_b, rotations):
    pos = np.arange(L, dtype=np.float32)[:, None]
    div = np.exp(np.arange(0, D, 2, dtype=np.float32) * (-np.log(10000.0) / D))
    pe_np = np.zeros((L, D), dtype=np.float32)
    pe_np[:, 0::2] = np.sin(pos * div)
    pe_np[:, 1::2] = np.cos(pos * div)
    pe = jnp.asarray(pe_np)

    val = jnp.roll(batch_x, 1, axis=1) @ conv_w[0] + batch_x @ conv_w[1] \
        + jnp.roll(batch_x, -1, axis=1) @ conv_w[2]
    x = (val + batch_x_time_stamp @ temp_w + pe[None]).reshape(M, D)

    pos_i = jnp.arange(L, dtype=jnp.int32)

    for l in range(NL):
        qkv = _qkv(x, jnp.concatenate([Wqk[l], Wv[l]], axis=1))
        qkh = qkv[:, :D].reshape(B, L, H, DH).transpose(0, 2, 1, 3)
        vh = qkv[:, D:].reshape(B, L, H, DH).transpose(0, 2, 1, 3)

        rot = rotations[l].transpose(1, 0, 2).reshape(DH, NH * (NBK // 2))
        bkt = _buckets(qkh, rot)

        skey = bkt * L + pos_i
        sticker = jnp.argsort(skey, axis=-1)
        undo = jnp.argsort(sticker, axis=-1)

        sqk = jnp.take_along_axis(qkh[:, :, None], sticker[..., None], axis=3)
        sv = jnp.take_along_axis(vh[:, :, None], sticker[..., None], axis=3)
        skn = sqk / (jnp.linalg.norm(sqk, axis=-1, keepdims=True) + 1e-9)

        so, slse = _attention(sqk.reshape(R4, L, DH), skn.reshape(R4, L, DH),
                              sv.reshape(R4, L, DH))
        so = so.reshape(B, H, NH, L, DH)
        slse = slse.reshape(B, H, NH, L)

        o_un = jnp.take_along_axis(so, undo[..., None], axis=3)
        lse_un = jnp.take_along_axis(slse, undo, axis=3)
        O = o_un.transpose(2, 0, 1, 3, 4)
        W = jax.nn.softmax(lse_un.transpose(2, 0, 1, 3), axis=0)[..., None]
        a = (O * W).sum(0)
        a = a.transpose(0, 2, 1, 3).reshape(M, D)

        ap = _mm_bias(a, Wo[l], bo[l])
        x = _layernorm(x + ap, ln1_g[l], ln1_b[l])
        h = jax.nn.gelu(_mm_bias(x, W1[l], b1[l]))
        y = _mm_bias(h, W2[l], b2[l])
        x = _layernorm(x + y, ln2_g[l], ln2_b[l])

    xn = _layernorm(x, lnf_g, lnf_b)
    pw = jnp.concatenate([proj_w, jnp.zeros((D, 128 - C_OUT), jnp.float32)],
                         axis=1)
    pb = jnp.concatenate([proj_b, jnp.zeros((128 - C_OUT,), jnp.float32)])
    out = _mm_bias(xn, pw, pb)
    return out[:, :C_OUT].reshape(B, L, C_OUT)


def kernel(batch_x, batch_x_time_stamp, conv_w, temp_w, Wqk, Wv, Wo, bo,
           ln1_g, ln1_b, W1, b1, W2, b2, ln2_g, ln2_b, lnf_g, lnf_b,
           proj_w, proj_b, rotations):
    return _forward_impl(batch_x, batch_x_time_stamp, conv_w, temp_w, Wqk,
                         Wv, Wo, bo, ln1_g, ln1_b, W1, b1, W2, b2, ln2_g,
                         ln2_b, lnf_g, lnf_b, proj_w, proj_b, rotations)
